# Initial kernel scaffold; baseline (speedup 1.0000x reference)
import jax
import jax.numpy as jnp
from jax.experimental import pallas as pl


def kernel(x, edge_index, W1, b1, W2, b2, W3, b3):
    raise NotImplementedError("write your pallas kernel here")


# speedup vs baseline: 6.1312x; 6.1312x over previous
import functools

import jax
import jax.numpy as jnp
from jax import lax
from jax.experimental import pallas as pl
from jax.experimental.pallas import tpu as pltpu
from jax.experimental.pallas import tpu_sc as plsc

N = 10000
E = 320000
NC = 2
NS = 16
NW = NC * NS
EPW = E // NW
C = 80
NCHUNK = EPW // C
NPAD = 10240
RPT = NPAD // NS
ZROWS = 128
DW = 16

_sc_mesh = plsc.VectorSubcoreMesh(core_axis_name="c", subcore_axis_name="s")


def _fill_rows(ref, nrows, width, value):
    v = jnp.full((16,), value, jnp.float32)

    def row(r, carry):
        for j in range(width // 16):
            ref[r, pl.ds(j * 16, 16)] = v
        return carry

    lax.fori_loop(0, nrows, row, 0)


def _make_agg(D):
    @functools.partial(
        pl.kernel,
        mesh=_sc_mesh,
        out_type=jax.ShapeDtypeStruct((NC, NPAD, D), jnp.float32),
        scratch_types=[
            pltpu.VMEM((C,), jnp.int32),
            pltpu.VMEM((C,), jnp.int32),
            pltpu.VMEM((C, D), jnp.float32),
            pltpu.VMEM((ZROWS, D), jnp.float32),
            pltpu.VMEM_SHARED((NPAD, D), jnp.float32),
            pltpu.SemaphoreType.DMA,
        ],
    )
    def agg(src_hbm, dst_hbm, hs_hbm, out_hbm, src_v, dst_v, rows_v, zbuf, acc, sem):
        cid = lax.axis_index("c")
        sid = lax.axis_index("s")
        wid = sid * NC + cid

        _fill_rows(zbuf, ZROWS, D, 0.0)
        for k in range(RPT // ZROWS):
            pltpu.sync_copy(zbuf, acc.at[pl.ds(sid * RPT + k * ZROWS, ZROWS)])
        plsc.subcore_barrier()

        base = wid * EPW

        def chunk(i, carry):
            off = base + i * C
            pltpu.sync_copy(src_hbm.at[pl.ds(off, C)], src_v)
            pltpu.sync_copy(dst_hbm.at[pl.ds(off, C)], dst_v)
            pltpu.async_copy(hs_hbm.at[src_v], rows_v, sem).wait()
            pltpu.sync_copy(rows_v, acc.at[dst_v], add=True)
            return carry

        lax.fori_loop(0, NCHUNK, chunk, 0)
        plsc.subcore_barrier()
        pltpu.sync_copy(acc.at[pl.ds(sid * RPT, RPT)],
                        out_hbm.at[cid, pl.ds(sid * RPT, RPT)])

    return agg


_agg128 = _make_agg(128)


@functools.partial(
    pl.kernel,
    mesh=_sc_mesh,
    out_type=jax.ShapeDtypeStruct((NC, NPAD, 128), jnp.float32),
    scratch_types=[
        pltpu.VMEM((C,), jnp.int32),
        pltpu.VMEM((C, 128), jnp.float32),
        pltpu.VMEM((ZROWS, 128), jnp.float32),
        pltpu.VMEM_SHARED((NPAD, 128), jnp.float32),
    ],
)
def _deg1(idx_hbm, out_hbm, idx_v, ones_v, zbuf, acc):
    cid = lax.axis_index("c")
    sid = lax.axis_index("s")
    wid = sid * NC + cid

    _fill_rows(ones_v, C, 128, 1.0)
    _fill_rows(zbuf, ZROWS, 128, 0.0)
    for k in range(RPT // ZROWS):
        pltpu.sync_copy(zbuf, acc.at[pl.ds(sid * RPT + k * ZROWS, ZROWS)])
    plsc.subcore_barrier()

    base = wid * EPW

    def chunk(i, carry):
        off = base + i * C
        pltpu.sync_copy(idx_hbm.at[pl.ds(off, C)], idx_v)
        pltpu.sync_copy(ones_v, acc.at[idx_v], add=True)
        return carry

    lax.fori_loop(0, NCHUNK, chunk, 0)
    plsc.subcore_barrier()
    pltpu.sync_copy(acc.at[pl.ds(sid * RPT, RPT)],
                    out_hbm.at[cid, pl.ds(sid * RPT, RPT)])


def _norms(degpo, degpi):
    def body(do_ref, di_ref, ns_ref, nd_ref):
        for d_ref, o_ref in ((do_ref, ns_ref), (di_ref, nd_ref)):
            d = (d_ref[0] + d_ref[1])[:N, :DW]
            o_ref[...] = jnp.where(d > 0.0, lax.rsqrt(jnp.maximum(d, 1.0)), 0.0)

    return pl.pallas_call(
        body,
        out_shape=(
            jax.ShapeDtypeStruct((N, DW), jnp.float32),
            jax.ShapeDtypeStruct((N, DW), jnp.float32),
        ),
    )(degpo, degpi)


def _mm_scale(h, W, ns):
    def body(h_ref, w_ref, s_ref, o_ref):
        o_ref[...] = jnp.dot(h_ref[...], w_ref[...],
                             preferred_element_type=jnp.float32) * s_ref[...][:, :1]

    return pl.pallas_call(
        body,
        out_shape=jax.ShapeDtypeStruct((N, W.shape[1]), jnp.float32),
    )(h, W, ns)


def _mid(p, nd, b, Wn, ns):
    def body(p_ref, nd_ref, b_ref, w_ref, ns_ref, o_ref):
        agg = (p_ref[0] + p_ref[1])[:N]
        h = agg * nd_ref[...][:, :1] + b_ref[...]
        h = jnp.maximum(h, 0.0)
        o_ref[...] = jnp.dot(h, w_ref[...],
                             preferred_element_type=jnp.float32) * ns_ref[...][:, :1]

    return pl.pallas_call(
        body,
        out_shape=jax.ShapeDtypeStruct((N, Wn.shape[1]), jnp.float32),
    )(p, nd, b, Wn, ns)


def _final(p, nd, b, dout):
    def body(p_ref, nd_ref, b_ref, o_ref):
        agg = (p_ref[0] + p_ref[1])[:N, :dout]
        o_ref[...] = agg * nd_ref[...][:, :1] + b_ref[...]

    return pl.pallas_call(
        body,
        out_shape=jax.ShapeDtypeStruct((N, dout), jnp.float32),
    )(p, nd, b)


def kernel(x, edge_index, W1, b1, W2, b2, W3, b3):
    src = edge_index[0].astype(jnp.int32)
    dst = edge_index[1].astype(jnp.int32)
    dout = W3.shape[1]
    b1 = b1.reshape(1, -1)
    b2 = b2.reshape(1, -1)
    b3 = b3.reshape(1, -1)
    W3p = jnp.pad(W3, ((0, 0), (0, 128 - dout)))

    degpo = _deg1(src)
    degpi = _deg1(dst)
    ns, nd = _norms(degpo, degpi)

    hs1 = _mm_scale(x, W1, ns)
    p1 = _agg128(src, dst, hs1)
    hs2 = _mid(p1, nd, b1, W2, ns)
    p2 = _agg128(src, dst, hs2)
    hs3 = _mid(p2, nd, b2, W3p, ns)
    p3 = _agg128(src, dst, hs3)
    return _final(p3, nd, b3, dout)
